# baseline (speedup 1.0000x reference)
import jax
import jax.numpy as jnp
from jax.experimental import pallas as pl
from jax.experimental.pallas import tpu as pltpu

BATCH = 4096
SEQ = 200
H_DIM = 32
ROW = SEQ * H_DIM
BB = 128
NCOPY = BATCH // BB


def _body(emb_ref, out_ref, scratch, sems):
    scratch[...] = jnp.broadcast_to(emb_ref[...], scratch.shape)
    for j in range(NCOPY):
        pltpu.make_async_copy(
            scratch, out_ref.at[pl.ds(j * BB, BB), :], sems.at[j]
        ).start()
    for j in range(NCOPY):
        pltpu.make_async_copy(
            scratch, out_ref.at[pl.ds(j * BB, BB), :], sems.at[j]
        ).wait()


def kernel(x, pos_embedding):
    del x
    emb_flat = pos_embedding[:SEQ].reshape(1, ROW)
    out = pl.pallas_call(
        _body,
        in_specs=[pl.BlockSpec((1, ROW), lambda: (0, 0))],
        out_specs=pl.BlockSpec(memory_space=pl.ANY),
        out_shape=jax.ShapeDtypeStruct((BATCH, ROW), jnp.float32),
        scratch_shapes=[
            pltpu.VMEM((BB, ROW), jnp.float32),
            pltpu.SemaphoreType.DMA((NCOPY,)),
        ],
    )(emb_flat)
    return out.reshape(BATCH, SEQ, H_DIM)
